# baseline (speedup 1.0000x reference)
import functools

import jax
import jax.numpy as jnp
from jax.experimental import pallas as pl
from jax.experimental.pallas import tpu as pltpu

_NE = 64
_K = 8
_BT = 512


def _body(x_ref, w_ref, gates_ref, idx_ref, aux_ref, acc_ref, *, n_tokens):
    i = pl.program_id(0)
    n = pl.num_programs(0)

    @pl.when(i == 0)
    def _():
        acc_ref[...] = jnp.zeros_like(acc_ref)

    logits = jnp.dot(x_ref[...], w_ref[...], preferred_element_type=jnp.float32)

    m = jnp.max(logits, axis=-1, keepdims=True)
    e = jnp.exp(logits - m)
    probs = e / jnp.sum(e, axis=-1, keepdims=True)
    psum = jnp.sum(probs, axis=0, keepdims=True)

    lane = jax.lax.broadcasted_iota(jnp.int32, logits.shape, 1)
    masked = logits
    vals, idxs = [], []
    for _ in range(_K):
        cm = jnp.max(masked, axis=-1, keepdims=True)
        hit = masked == cm
        am = jnp.min(jnp.where(hit, lane, _NE), axis=-1, keepdims=True)
        vals.append(cm)
        idxs.append(am)
        masked = jnp.where(lane == am, -jnp.inf, masked)
    topv = jnp.concatenate(vals, axis=-1)
    topi = jnp.concatenate(idxs, axis=-1)

    ge = jnp.exp(topv - topv[:, 0:1])
    gates_ref[...] = ge / jnp.sum(ge, axis=-1, keepdims=True)
    idx_ref[...] = topi

    fsum = jnp.sum((lane == topi[:, 0:1]).astype(jnp.float32), axis=0,
                   keepdims=True)
    acc_ref[0:1, :] += fsum
    acc_ref[1:2, :] += psum

    @pl.when(i == n - 1)
    def _():
        inv = jnp.float32(1.0 / n_tokens)
        f = acc_ref[0:1, :] * inv
        pmean = acc_ref[1:2, :] * inv
        aux_ref[0, 0] = _NE * jnp.sum(f * pmean)


def kernel(x, W):
    b, s, d = x.shape
    t = b * s
    x2 = x.reshape(t, d)
    grid = t // _BT
    gates, idx, aux = pl.pallas_call(
        functools.partial(_body, n_tokens=t),
        grid=(grid,),
        in_specs=[
            pl.BlockSpec((_BT, d), lambda i: (i, 0)),
            pl.BlockSpec((d, _NE), lambda i: (0, 0)),
        ],
        out_specs=[
            pl.BlockSpec((_BT, _K), lambda i: (i, 0)),
            pl.BlockSpec((_BT, _K), lambda i: (i, 0)),
            pl.BlockSpec(memory_space=pltpu.SMEM),
        ],
        out_shape=[
            jax.ShapeDtypeStruct((t, _K), jnp.float32),
            jax.ShapeDtypeStruct((t, _K), jnp.int32),
            jax.ShapeDtypeStruct((1, 1), jnp.float32),
        ],
        scratch_shapes=[pltpu.VMEM((2, _NE), jnp.float32)],
        compiler_params=pltpu.CompilerParams(
            dimension_semantics=("arbitrary",),
        ),
    )(x2, W)
    return gates.reshape(b, s, _K), idx.reshape(b, s, _K), aux[0, 0]


# speedup vs baseline: 1.8200x; 1.8200x over previous
import functools

import jax
import jax.numpy as jnp
from jax.experimental import pallas as pl
from jax.experimental.pallas import tpu as pltpu

_NE = 64
_K = 8
_BT = 512
_LG = 128


def _body(wt_ref, x_ref, gates_ref, idx_ref, aux_ref, acc_ref, *, n_tokens):
    i = pl.program_id(0)
    n = pl.num_programs(0)

    @pl.when(i == 0)
    def _():
        acc_ref[...] = jnp.zeros_like(acc_ref)

    logits = jax.lax.dot_general(
        wt_ref[...], x_ref[...],
        dimension_numbers=(((1,), (1,)), ((), ())),
        preferred_element_type=jnp.float32)

    sub = jax.lax.broadcasted_iota(jnp.int32, (_NE, _BT), 0)

    m = jnp.max(logits, axis=0, keepdims=True)
    e = jnp.exp(logits - m)
    probs = e / jnp.sum(e, axis=0, keepdims=True)

    masked = logits
    vals, idxs = [], []
    for _ in range(_K):
        cm = jnp.max(masked, axis=0, keepdims=True)
        hit = masked == cm
        am = jnp.min(jnp.where(hit, sub, _NE), axis=0, keepdims=True)
        vals.append(cm)
        idxs.append(am)
        masked = jnp.where(sub == am, -jnp.inf, masked)
    topv = jnp.concatenate(vals, axis=0)
    topi = jnp.concatenate(idxs, axis=0)

    ge = jnp.exp(topv - vals[0])
    gates_ref[...] = ge / jnp.sum(ge, axis=0, keepdims=True)
    idx_ref[...] = topi

    onehot = (sub == idxs[0]).astype(jnp.float32)
    fpart = jnp.zeros((_NE, _LG), jnp.float32)
    ppart = jnp.zeros((_NE, _LG), jnp.float32)
    for j in range(_BT // _LG):
        fpart = fpart + onehot[:, j * _LG:(j + 1) * _LG]
        ppart = ppart + probs[:, j * _LG:(j + 1) * _LG]
    acc_ref[0] += fpart
    acc_ref[1] += ppart

    @pl.when(i == n - 1)
    def _():
        inv = jnp.float32(1.0 / n_tokens)
        f = jnp.sum(acc_ref[0], axis=1, keepdims=True) * inv
        pmean = jnp.sum(acc_ref[1], axis=1, keepdims=True) * inv
        aux_ref[0, 0] = _NE * jnp.sum(f * pmean)


def kernel(x, W):
    b, s, d = x.shape
    t = b * s
    x2 = x.reshape(t, d)
    wt = W.T
    grid = t // _BT
    gates_t, idx_t, aux = pl.pallas_call(
        functools.partial(_body, n_tokens=t),
        grid=(grid,),
        in_specs=[
            pl.BlockSpec((_NE, d), lambda i: (0, 0)),
            pl.BlockSpec((_BT, d), lambda i: (i, 0)),
        ],
        out_specs=[
            pl.BlockSpec((_K, _BT), lambda i: (0, i)),
            pl.BlockSpec((_K, _BT), lambda i: (0, i)),
            pl.BlockSpec(memory_space=pltpu.SMEM),
        ],
        out_shape=[
            jax.ShapeDtypeStruct((_K, t), jnp.float32),
            jax.ShapeDtypeStruct((_K, t), jnp.int32),
            jax.ShapeDtypeStruct((1, 1), jnp.float32),
        ],
        scratch_shapes=[pltpu.VMEM((2, _NE, _LG), jnp.float32)],
        compiler_params=pltpu.CompilerParams(
            dimension_semantics=("arbitrary",),
        ),
    )(wt, x2)
    gates = gates_t.T.reshape(b, s, _K)
    idx = idx_t.T.reshape(b, s, _K)
    return gates, idx, aux[0, 0]


# speedup vs baseline: 2.0212x; 1.1106x over previous
import functools

import jax
import jax.numpy as jnp
from jax.experimental import pallas as pl
from jax.experimental.pallas import tpu as pltpu

_NE = 64
_K = 8
_BT = 1024
_LG = 128


def _body(wt_ref, x_ref, gates_ref, idx_ref, aux_ref, acc_ref, *, n_tokens):
    i = pl.program_id(0)
    n = pl.num_programs(0)

    @pl.when(i == 0)
    def _():
        acc_ref[...] = jnp.zeros_like(acc_ref)

    logits = jax.lax.dot_general(
        wt_ref[...], x_ref[...],
        dimension_numbers=(((1,), (1,)), ((), ())),
        preferred_element_type=jnp.float32)

    sub = jax.lax.broadcasted_iota(jnp.int32, (_NE, _BT), 0)

    m = jnp.max(logits, axis=0, keepdims=True)
    e = jnp.exp(logits - m)
    probs = e / jnp.sum(e, axis=0, keepdims=True)

    masked = logits
    vals, idxs = [], []
    for _ in range(_K):
        cm = jnp.max(masked, axis=0, keepdims=True)
        hit = masked == cm
        am = jnp.min(jnp.where(hit, sub, _NE), axis=0, keepdims=True)
        vals.append(cm)
        idxs.append(am)
        masked = jnp.where(sub == am, -jnp.inf, masked)
    topv = jnp.concatenate(vals, axis=0)
    topi = jnp.concatenate(idxs, axis=0)

    ge = jnp.exp(topv - vals[0])
    gates_ref[...] = ge / jnp.sum(ge, axis=0, keepdims=True)
    idx_ref[...] = topi

    onehot = (sub == idxs[0]).astype(jnp.float32)
    fpart = jnp.zeros((_NE, _LG), jnp.float32)
    ppart = jnp.zeros((_NE, _LG), jnp.float32)
    for j in range(_BT // _LG):
        fpart = fpart + onehot[:, j * _LG:(j + 1) * _LG]
        ppart = ppart + probs[:, j * _LG:(j + 1) * _LG]
    acc_ref[0] += fpart
    acc_ref[1] += ppart

    @pl.when(i == n - 1)
    def _():
        inv = jnp.float32(1.0 / n_tokens)
        f = jnp.sum(acc_ref[0], axis=1, keepdims=True) * inv
        pmean = jnp.sum(acc_ref[1], axis=1, keepdims=True) * inv
        aux_ref[0, 0] = _NE * jnp.sum(f * pmean)


def kernel(x, W):
    b, s, d = x.shape
    t = b * s
    x2 = x.reshape(t, d)
    wt = W.T
    grid = t // _BT
    gates_t, idx_t, aux = pl.pallas_call(
        functools.partial(_body, n_tokens=t),
        grid=(grid,),
        in_specs=[
            pl.BlockSpec((_NE, d), lambda i: (0, 0)),
            pl.BlockSpec((_BT, d), lambda i: (i, 0)),
        ],
        out_specs=[
            pl.BlockSpec((_K, _BT), lambda i: (0, i)),
            pl.BlockSpec((_K, _BT), lambda i: (0, i)),
            pl.BlockSpec(memory_space=pltpu.SMEM),
        ],
        out_shape=[
            jax.ShapeDtypeStruct((_K, t), jnp.float32),
            jax.ShapeDtypeStruct((_K, t), jnp.int32),
            jax.ShapeDtypeStruct((1, 1), jnp.float32),
        ],
        scratch_shapes=[pltpu.VMEM((2, _NE, _LG), jnp.float32)],
        compiler_params=pltpu.CompilerParams(
            dimension_semantics=("arbitrary",),
        ),
    )(wt, x2)
    gates = gates_t.T.reshape(b, s, _K)
    idx = idx_t.T.reshape(b, s, _K)
    return gates, idx, aux[0, 0]
